# baseline (speedup 1.0000x reference)
import jax
import jax.numpy as jnp
from jax import lax
from jax.experimental import pallas as pl
from jax.experimental.pallas import tpu as pltpu
from jax.experimental.pallas import tpu_sc as plsc

_MAX_ATOMS = 13
_T = 256
_NC = 2
_NS = 16
_NW = _NC * _NS
_IW = 80


def _grouped_matmul_body(lohi_ref, ids_ref, rn_ref, w_ref, out_ref):
    out_ref[...] = jnp.zeros_like(out_ref)
    lo = lohi_ref[0, 0, 0]
    hi = lohi_ref[0, 0, 1]

    def body(p, carry):
        m = (ids_ref[0] == p).astype(jnp.float32)
        out_ref[...] += jnp.dot(rn_ref[...] * m, w_ref[p],
                                preferred_element_type=jnp.float32)
        return carry

    lax.fori_loop(lo, hi + 1, body, 0)
    out_ref[...] = jnp.maximum(out_ref[...], 0.0)


def kernel(x, nodes, fact, fact_dim, params, bias):
    F = fact.shape[0]
    N, L = nodes.shape
    P, _, R = params.shape
    fact = fact.astype(jnp.int32)

    ids = (x[fact[:, 0], 1].astype(jnp.int32) * _MAX_ATOMS
           + x[fact[:, 0], 2].astype(jnp.int32))
    ids_s, order = lax.sort_key_val(ids, jnp.arange(F, dtype=jnp.int32))

    cbj = pl.cdiv(pl.cdiv(F, _NW), _IW) * _IW
    rtot = _NW * 2 * cbj
    tiles = rtot // _T
    nck = cbj // _IW
    jpad = _NW * cbj - F
    assert rtot % _T == 0

    order_pad = jnp.pad(order, (0, jpad), mode="edge")
    jr = jnp.arange(_NW * cbj, dtype=jnp.int32)
    d0c = (2 * jr).reshape(_NW * nck, _IW)
    d1c = (2 * jr + 1).reshape(_NW * nck, _IW)
    u0c = order_pad.reshape(_NW * nck, _IW)
    u1c = (order_pad + F).reshape(_NW * nck, _IW)

    fact0 = fact[:, 0]
    fact1 = fact[:, 1]
    nodes_aug = jnp.concatenate(
        [nodes, jnp.ones((N, 1), nodes.dtype),
         jnp.zeros((N, R - L - 1), nodes.dtype)], axis=1)
    w_aug = jnp.concatenate(
        [params, bias, jnp.zeros((P, R - L - 1, R), params.dtype)], axis=1)
    mesh = plsc.VectorSubcoreMesh(core_axis_name="c", subcore_axis_name="s")

    def sc_gather(order_hbm, d0_hbm, d1_hbm, fact0_hbm, fact1_hbm, nodes_hbm,
                  rn_hbm,
                  o_v, d0_v, d1_v, f0_v, f1_v, rows0_v, rows1_v, s0, s1):
        wid = lax.axis_index("s") * _NC + lax.axis_index("c")
        pltpu.sync_copy(order_hbm.at[pl.ds(wid * cbj, cbj)], o_v)
        pltpu.sync_copy(d0_hbm.at[pl.ds(wid * nck, nck)], d0_v)
        pltpu.sync_copy(d1_hbm.at[pl.ds(wid * nck, nck)], d1_v)
        c0 = pltpu.async_copy(fact0_hbm.at[o_v], f0_v, s0)
        c1 = pltpu.async_copy(fact1_hbm.at[o_v], f1_v, s1)
        c0.wait()
        c1.wait()
        g0 = pltpu.async_copy(nodes_hbm.at[f0_v], rows0_v, s0)
        g1 = pltpu.async_copy(nodes_hbm.at[f1_v], rows1_v, s1)
        g0.wait()
        g1.wait()
        for c in range(nck):
            pltpu.async_copy(rows0_v.at[pl.ds(c * _IW, _IW)],
                             rn_hbm.at[d0_v.at[c]], s0)
            pltpu.async_copy(rows1_v.at[pl.ds(c * _IW, _IW)],
                             rn_hbm.at[d1_v.at[c]], s1)
        for c in range(nck):
            pltpu.make_async_copy(rows0_v.at[pl.ds(c * _IW, _IW)],
                                  rn_hbm.at[d0_v.at[c]], s0).wait()
            pltpu.make_async_copy(rows1_v.at[pl.ds(c * _IW, _IW)],
                                  rn_hbm.at[d1_v.at[c]], s1).wait()

    rn = pl.kernel(
        sc_gather,
        out_type=jax.ShapeDtypeStruct((rtot, R), jnp.float32),
        mesh=mesh,
        scratch_types=[
            pltpu.VMEM((cbj,), jnp.int32),
            pltpu.VMEM((nck, _IW), jnp.int32),
            pltpu.VMEM((nck, _IW), jnp.int32),
            pltpu.VMEM((cbj,), jnp.int32),
            pltpu.VMEM((cbj,), jnp.int32),
            pltpu.VMEM((cbj, R), jnp.float32),
            pltpu.VMEM((cbj, R), jnp.float32),
            pltpu.SemaphoreType.DMA,
            pltpu.SemaphoreType.DMA,
        ],
    )(order_pad, d0c, d1c, fact0, fact1, nodes_aug)

    ids_rep = jnp.repeat(jnp.pad(ids_s, (0, jpad), mode="edge"), 2)
    lohi = jnp.stack([ids_rep[::_T], ids_rep[_T - 1::_T]],
                     axis=1).astype(jnp.int32).reshape(tiles, 1, 2)
    ids_b = ids_rep.reshape(tiles, _T, 1).astype(jnp.int32)

    out_sorted = pl.pallas_call(
        _grouped_matmul_body,
        grid=(tiles,),
        in_specs=[
            pl.BlockSpec((1, 1, 2), lambda i: (i, 0, 0), memory_space=pltpu.SMEM),
            pl.BlockSpec((1, _T, 1), lambda i: (i, 0, 0)),
            pl.BlockSpec((_T, R), lambda i: (i, 0)),
            pl.BlockSpec((P, R, R), lambda i: (0, 0, 0)),
        ],
        out_specs=pl.BlockSpec((_T, R), lambda i: (i, 0)),
        out_shape=jax.ShapeDtypeStruct((rtot, R), jnp.float32),
    )(lohi, ids_b, rn, w_aug)

    s0c = (2 * jr).reshape(_NW * nck, _IW)
    s1c = (2 * jr + 1).reshape(_NW * nck, _IW)

    def sc_unsort2(outs_hbm, s0_hbm, s1_hbm, u0_hbm, u1_hbm, final_hbm,
                   s0_v, s1_v, u0_v, u1_v, rows_v, d0, d1):
        wid = lax.axis_index("s") * _NC + lax.axis_index("c")
        pltpu.sync_copy(s0_hbm.at[pl.ds(wid * nck, nck)], s0_v)
        pltpu.sync_copy(s1_hbm.at[pl.ds(wid * nck, nck)], s1_v)
        pltpu.sync_copy(u0_hbm.at[pl.ds(wid * nck, nck)], u0_v)
        pltpu.sync_copy(u1_hbm.at[pl.ds(wid * nck, nck)], u1_v)
        for c in range(nck):
            pltpu.async_copy(outs_hbm.at[s0_v.at[c]],
                             rows_v.at[pl.ds(c * _IW, _IW)], d0)
            pltpu.async_copy(outs_hbm.at[s1_v.at[c]],
                             rows_v.at[pl.ds(cbj + c * _IW, _IW)], d1)
        for c in range(nck):
            pltpu.make_async_copy(outs_hbm.at[s0_v.at[c]],
                                  rows_v.at[pl.ds(c * _IW, _IW)], d0).wait()
            pltpu.make_async_copy(outs_hbm.at[s1_v.at[c]],
                                  rows_v.at[pl.ds(cbj + c * _IW, _IW)], d1).wait()
        for c in range(nck):
            pltpu.async_copy(rows_v.at[pl.ds(c * _IW, _IW)],
                             final_hbm.at[u0_v.at[c]], d0)
            pltpu.async_copy(rows_v.at[pl.ds(cbj + c * _IW, _IW)],
                             final_hbm.at[u1_v.at[c]], d1)
        for c in range(nck):
            pltpu.make_async_copy(rows_v.at[pl.ds(c * _IW, _IW)],
                                  final_hbm.at[u0_v.at[c]], d0).wait()
            pltpu.make_async_copy(rows_v.at[pl.ds(cbj + c * _IW, _IW)],
                                  final_hbm.at[u1_v.at[c]], d1).wait()

    final = pl.kernel(
        sc_unsort2,
        out_type=jax.ShapeDtypeStruct((2 * F, R), jnp.float32),
        mesh=mesh,
        scratch_types=[
            pltpu.VMEM((nck, _IW), jnp.int32),
            pltpu.VMEM((nck, _IW), jnp.int32),
            pltpu.VMEM((nck, _IW), jnp.int32),
            pltpu.VMEM((nck, _IW), jnp.int32),
            pltpu.VMEM((2 * cbj, R), jnp.float32),
            pltpu.SemaphoreType.DMA,
            pltpu.SemaphoreType.DMA,
        ],
    )(out_sorted, s0c, s1c, u0c, u1c)

    return final.reshape(2, F, R)


# speedup vs baseline: 1.1153x; 1.1153x over previous
import jax
import jax.numpy as jnp
from jax import lax
from jax.experimental import pallas as pl
from jax.experimental.pallas import tpu as pltpu
from jax.experimental.pallas import tpu_sc as plsc

_MAX_ATOMS = 13
_T = 512
_NC = 2
_NS = 16
_NW = _NC * _NS
_IW = 80


def _grouped_matmul_body(lohi_ref, ids_ref, rn_ref, w_ref, out_ref):
    out_ref[...] = jnp.zeros_like(out_ref)
    lo = lohi_ref[0, 0, 0]
    hi = lohi_ref[0, 0, 1]

    def body(p, carry):
        m = (ids_ref[0] == p).astype(jnp.float32)
        out_ref[...] += jnp.dot(rn_ref[...] * m, w_ref[p],
                                preferred_element_type=jnp.float32)
        return carry

    lax.fori_loop(lo, hi + 1, body, 0)
    out_ref[...] = jnp.maximum(out_ref[...], 0.0)


def kernel(x, nodes, fact, fact_dim, params, bias):
    F = fact.shape[0]
    N, L = nodes.shape
    P, _, R = params.shape
    fact = fact.astype(jnp.int32)

    ids = (x[fact[:, 0], 1].astype(jnp.int32) * _MAX_ATOMS
           + x[fact[:, 0], 2].astype(jnp.int32))
    ids_s, order = lax.sort_key_val(ids, jnp.arange(F, dtype=jnp.int32))

    cbj = pl.cdiv(pl.cdiv(F, _NW), _IW) * _IW
    rtot = _NW * 2 * cbj
    tiles = rtot // _T
    nck = cbj // _IW
    jpad = _NW * cbj - F
    assert rtot % _T == 0

    order_pad = jnp.pad(order, (0, jpad), mode="edge")
    jr = jnp.arange(_NW * cbj, dtype=jnp.int32)
    d0c = (2 * jr).reshape(_NW * nck, _IW)
    d1c = (2 * jr + 1).reshape(_NW * nck, _IW)
    u0c = order_pad.reshape(_NW * nck, _IW)
    u1c = (order_pad + F).reshape(_NW * nck, _IW)

    fact0 = fact[:, 0]
    fact1 = fact[:, 1]
    nodes_aug = jnp.concatenate(
        [nodes, jnp.ones((N, 1), nodes.dtype),
         jnp.zeros((N, R - L - 1), nodes.dtype)], axis=1)
    w_aug = jnp.concatenate(
        [params, bias, jnp.zeros((P, R - L - 1, R), params.dtype)], axis=1)
    mesh = plsc.VectorSubcoreMesh(core_axis_name="c", subcore_axis_name="s")

    def sc_gather(order_hbm, d0_hbm, d1_hbm, fact0_hbm, fact1_hbm, nodes_hbm,
                  rn_hbm,
                  o_v, d0_v, d1_v, f0_v, f1_v, rows0_v, rows1_v, s0, s1):
        wid = lax.axis_index("s") * _NC + lax.axis_index("c")
        pltpu.sync_copy(order_hbm.at[pl.ds(wid * cbj, cbj)], o_v)
        pltpu.sync_copy(d0_hbm.at[pl.ds(wid * nck, nck)], d0_v)
        pltpu.sync_copy(d1_hbm.at[pl.ds(wid * nck, nck)], d1_v)
        c0 = pltpu.async_copy(fact0_hbm.at[o_v], f0_v, s0)
        c1 = pltpu.async_copy(fact1_hbm.at[o_v], f1_v, s1)
        c0.wait()
        c1.wait()
        g0 = pltpu.async_copy(nodes_hbm.at[f0_v], rows0_v, s0)
        g1 = pltpu.async_copy(nodes_hbm.at[f1_v], rows1_v, s1)
        g0.wait()
        g1.wait()
        for c in range(nck):
            pltpu.async_copy(rows0_v.at[pl.ds(c * _IW, _IW)],
                             rn_hbm.at[d0_v.at[c]], s0)
            pltpu.async_copy(rows1_v.at[pl.ds(c * _IW, _IW)],
                             rn_hbm.at[d1_v.at[c]], s1)
        for c in range(nck):
            pltpu.make_async_copy(rows0_v.at[pl.ds(c * _IW, _IW)],
                                  rn_hbm.at[d0_v.at[c]], s0).wait()
            pltpu.make_async_copy(rows1_v.at[pl.ds(c * _IW, _IW)],
                                  rn_hbm.at[d1_v.at[c]], s1).wait()

    rn = pl.kernel(
        sc_gather,
        out_type=jax.ShapeDtypeStruct((rtot, R), jnp.float32),
        mesh=mesh,
        scratch_types=[
            pltpu.VMEM((cbj,), jnp.int32),
            pltpu.VMEM((nck, _IW), jnp.int32),
            pltpu.VMEM((nck, _IW), jnp.int32),
            pltpu.VMEM((cbj,), jnp.int32),
            pltpu.VMEM((cbj,), jnp.int32),
            pltpu.VMEM((cbj, R), jnp.float32),
            pltpu.VMEM((cbj, R), jnp.float32),
            pltpu.SemaphoreType.DMA,
            pltpu.SemaphoreType.DMA,
        ],
    )(order_pad, d0c, d1c, fact0, fact1, nodes_aug)

    ids_rep = jnp.repeat(jnp.pad(ids_s, (0, jpad), mode="edge"), 2)
    lohi = jnp.stack([ids_rep[::_T], ids_rep[_T - 1::_T]],
                     axis=1).astype(jnp.int32).reshape(tiles, 1, 2)
    ids_b = ids_rep.reshape(tiles, _T, 1).astype(jnp.int32)

    out_sorted = pl.pallas_call(
        _grouped_matmul_body,
        grid=(tiles,),
        in_specs=[
            pl.BlockSpec((1, 1, 2), lambda i: (i, 0, 0), memory_space=pltpu.SMEM),
            pl.BlockSpec((1, _T, 1), lambda i: (i, 0, 0)),
            pl.BlockSpec((_T, R), lambda i: (i, 0)),
            pl.BlockSpec((P, R, R), lambda i: (0, 0, 0)),
        ],
        out_specs=pl.BlockSpec((_T, R), lambda i: (i, 0)),
        out_shape=jax.ShapeDtypeStruct((rtot, R), jnp.float32),
    )(lohi, ids_b, rn, w_aug)

    s0c = (2 * jr).reshape(_NW * nck, _IW)
    s1c = (2 * jr + 1).reshape(_NW * nck, _IW)

    def sc_unsort2(outs_hbm, s0_hbm, s1_hbm, u0_hbm, u1_hbm, final_hbm,
                   s0_v, s1_v, u0_v, u1_v, rows_v, d0, d1):
        wid = lax.axis_index("s") * _NC + lax.axis_index("c")
        pltpu.sync_copy(s0_hbm.at[pl.ds(wid * nck, nck)], s0_v)
        pltpu.sync_copy(s1_hbm.at[pl.ds(wid * nck, nck)], s1_v)
        pltpu.sync_copy(u0_hbm.at[pl.ds(wid * nck, nck)], u0_v)
        pltpu.sync_copy(u1_hbm.at[pl.ds(wid * nck, nck)], u1_v)
        for c in range(nck):
            pltpu.async_copy(outs_hbm.at[s0_v.at[c]],
                             rows_v.at[pl.ds(c * _IW, _IW)], d0)
            pltpu.async_copy(outs_hbm.at[s1_v.at[c]],
                             rows_v.at[pl.ds(cbj + c * _IW, _IW)], d1)
        for c in range(nck):
            pltpu.make_async_copy(outs_hbm.at[s0_v.at[c]],
                                  rows_v.at[pl.ds(c * _IW, _IW)], d0).wait()
            pltpu.make_async_copy(outs_hbm.at[s1_v.at[c]],
                                  rows_v.at[pl.ds(cbj + c * _IW, _IW)], d1).wait()
        for c in range(nck):
            pltpu.async_copy(rows_v.at[pl.ds(c * _IW, _IW)],
                             final_hbm.at[u0_v.at[c]], d0)
            pltpu.async_copy(rows_v.at[pl.ds(cbj + c * _IW, _IW)],
                             final_hbm.at[u1_v.at[c]], d1)
        for c in range(nck):
            pltpu.make_async_copy(rows_v.at[pl.ds(c * _IW, _IW)],
                                  final_hbm.at[u0_v.at[c]], d0).wait()
            pltpu.make_async_copy(rows_v.at[pl.ds(cbj + c * _IW, _IW)],
                                  final_hbm.at[u1_v.at[c]], d1).wait()

    final = pl.kernel(
        sc_unsort2,
        out_type=jax.ShapeDtypeStruct((2 * F, R), jnp.float32),
        mesh=mesh,
        scratch_types=[
            pltpu.VMEM((nck, _IW), jnp.int32),
            pltpu.VMEM((nck, _IW), jnp.int32),
            pltpu.VMEM((nck, _IW), jnp.int32),
            pltpu.VMEM((nck, _IW), jnp.int32),
            pltpu.VMEM((2 * cbj, R), jnp.float32),
            pltpu.SemaphoreType.DMA,
            pltpu.SemaphoreType.DMA,
        ],
    )(out_sorted, s0c, s1c, u0c, u1c)

    return final.reshape(2, F, R)
